# Initial kernel scaffold; baseline (speedup 1.0000x reference)
import jax
import jax.numpy as jnp
from jax.experimental import pallas as pl


def kernel(x, Wq, Wk, value_experts, output_experts, gate_w):
    raise NotImplementedError("write your pallas kernel here")


# speedup vs baseline: 1.2141x; 1.2141x over previous
import jax
import jax.numpy as jnp
from jax.experimental import pallas as pl

N = 2048
D = 768
H = 12
DH = 64
E = 8
BQ = 512


def _head_body(x_ref, wq_ref, wk_ref, wg_ref, wv_ref, wo_ref, out_ref):
    h = pl.program_id(0)
    X = x_ref[...]
    q = jnp.dot(X, wq_ref[0], preferred_element_type=jnp.float32)
    k = jnp.dot(X, wk_ref[0], preferred_element_type=jnp.float32)
    logits = jnp.dot(X, wg_ref[0], preferred_element_type=jnp.float32)

    m = jnp.max(logits, axis=-1, keepdims=True)
    col = jax.lax.broadcasted_iota(jnp.int32, (N, E), 1)
    e_idx = jnp.min(jnp.where(logits == m, col, E), axis=-1, keepdims=True)
    gate = jax.nn.sigmoid(m)

    pv = jnp.dot(X, wv_ref[0], preferred_element_type=jnp.float32)
    vals = jnp.zeros((N, DH), jnp.float32)
    for ex in range(E):
        sel = jnp.where(e_idx == ex, gate, 0.0)
        vals = vals + sel * pv[:, ex * DH:(ex + 1) * DH]

    scale = DH ** -0.5
    a_blocks = []
    for i in range(N // BQ):
        W = (i + 1) * BQ
        qi = q[i * BQ:(i + 1) * BQ] * scale
        s = jax.lax.dot_general(qi, k[:W], (((1,), (1,)), ((), ())),
                                preferred_element_type=jnp.float32)
        rows = jax.lax.broadcasted_iota(jnp.int32, (BQ, W), 0) + i * BQ
        cols = jax.lax.broadcasted_iota(jnp.int32, (BQ, W), 1)
        s = jnp.where(cols > rows, -jnp.inf, s)
        s = s - jnp.max(s, axis=-1, keepdims=True)
        p = jnp.exp(s)
        p = p / jnp.sum(p, axis=-1, keepdims=True)
        a_blocks.append(jnp.dot(p, vals[:W], preferred_element_type=jnp.float32))
    a = jnp.concatenate(a_blocks, axis=0) * gate

    expand = jnp.concatenate(
        [jnp.where(e_idx == ex, a, 0.0) for ex in range(E)], axis=1)
    contrib = jnp.dot(expand, wo_ref[0], preferred_element_type=jnp.float32)

    @pl.when(h == 0)
    def _():
        out_ref[...] = contrib

    @pl.when(h != 0)
    def _():
        out_ref[...] = out_ref[...] + contrib


@jax.jit
def kernel(x, Wq, Wk, value_experts, output_experts, gate_w):
    xs = x[0]
    wq = Wq.reshape(D, H, DH).transpose(1, 0, 2)
    wk = Wk.reshape(D, H, DH).transpose(1, 0, 2)
    wg = gate_w.reshape(D, H, E).transpose(1, 0, 2)
    wv = value_experts.transpose(1, 2, 0, 3).reshape(H, D, E * DH)
    wo = output_experts.transpose(1, 0, 2, 3).reshape(H, E * DH, D)
    out = pl.pallas_call(
        _head_body,
        grid=(H,),
        in_specs=[
            pl.BlockSpec((N, D), lambda h: (0, 0)),
            pl.BlockSpec((1, D, DH), lambda h: (h, 0, 0)),
            pl.BlockSpec((1, D, DH), lambda h: (h, 0, 0)),
            pl.BlockSpec((1, D, E), lambda h: (h, 0, 0)),
            pl.BlockSpec((1, D, E * DH), lambda h: (h, 0, 0)),
            pl.BlockSpec((1, E * DH, D), lambda h: (h, 0, 0)),
        ],
        out_specs=pl.BlockSpec((N, D), lambda h: (0, 0)),
        out_shape=jax.ShapeDtypeStruct((N, D), jnp.float32),
    )(xs, wq, wk, wg, wv, wo)
    return out[None]
